# Initial kernel scaffold; baseline (speedup 1.0000x reference)
import jax
import jax.numpy as jnp
from jax.experimental import pallas as pl


def kernel(y_pred, y_true, interval_weights, bin_edges, conditional_info):
    raise NotImplementedError("write your pallas kernel here")


# speedup vs baseline: 388.5074x; 388.5074x over previous
import functools

import jax
import jax.numpy as jnp
from jax import lax
from jax.experimental import pallas as pl
from jax.experimental.pallas import tpu as pltpu
from jax.experimental.pallas import tpu_sc as plsc

_NC = 2
_NS = 16
_NW = _NC * _NS
_L = 16

_BUF = 2000
_VPC = _BUF // _L


def _sc_binning(n_padded: int, n_bins: int):
    chunk = n_padded // _NW
    n_chunks = chunk // _BUF
    assert chunk % _BUF == 0 and n_padded % _NW == 0
    steps = []
    s = n_bins
    while s > 1:
        s //= 2
        steps.append(s)
    assert (1 << len(steps)) == n_bins

    mesh = plsc.VectorSubcoreMesh(core_axis_name="c", subcore_axis_name="s")

    @functools.partial(
        pl.kernel,
        mesh=mesh,
        out_type=[
            jax.ShapeDtypeStruct((n_padded,), jnp.float32),
            jax.ShapeDtypeStruct((_NW, _L), jnp.float32),
        ],
        compiler_params=pltpu.CompilerParams(needs_layout_passes=False),
        scratch_types=[
            pltpu.VMEM((_BUF,), jnp.float32),
            pltpu.VMEM((_BUF,), jnp.float32),
            pltpu.VMEM((n_bins,), jnp.float32),
            pltpu.VMEM((n_bins,), jnp.float32),
            pltpu.VMEM((_L,), jnp.float32),
        ],
    )
    def sc_kernel(cond_hbm, edges_hbm, iw_hbm, w_out, minfail_out,
                  xbuf, wbuf, edges_v, v_v, min_v):
        wid = lax.axis_index("c") * _NS + lax.axis_index("s")
        base = wid * chunk

        pltpu.sync_copy(edges_hbm, edges_v)
        pltpu.sync_copy(iw_hbm, v_v)
        for k in range(n_bins // _L):
            sl = pl.ds(k * _L, _L)
            v_v[sl] = jnp.maximum(v_v[sl], 0.01)

        big = jnp.full((_L,), jnp.inf, jnp.float32)

        def vec_body(i, minacc):
            x = xbuf[pl.ds(i * _L, _L)]
            pos = jnp.zeros((_L,), jnp.int32)
            for s in steps:
                probe = pos + (s - 1)
                e = plsc.load_gather(edges_v, [probe])
                pos = jnp.where(e < x, pos + s, pos)
            ej = plsc.load_gather(edges_v, [jnp.minimum(pos, n_bins - 1)])
            fail = (pos == 0) | (pos == n_bins) | (ej == x)
            fail = fail & (x >= 0.0)
            minacc = jnp.minimum(minacc, jnp.where(fail, x, big))
            widx = jnp.clip(pos - 1, 0, n_bins - 1)
            wbuf[pl.ds(i * _L, _L)] = plsc.load_gather(v_v, [widx])
            return minacc

        def chunk_body(k, minacc):
            g = base + k * _BUF
            pltpu.sync_copy(cond_hbm.at[pl.ds(g, _BUF)], xbuf)
            minacc = lax.fori_loop(0, _VPC, vec_body, minacc)
            pltpu.sync_copy(wbuf, w_out.at[pl.ds(g, _BUF)])
            return minacc

        minacc = lax.fori_loop(0, n_chunks, chunk_body, big)
        min_v[...] = minacc
        pltpu.sync_copy(min_v, minfail_out.at[wid])

    return sc_kernel


def _tc_loss(n_real: int, n_bins: int, rows: int, block_rows: int):
    grid = rows // block_rows
    assert rows % block_rows == 0

    def body(tmin_ref, iw_ref, w_ref, c_ref, yp_ref, yt_ref, out_ref):
        i = pl.program_id(0)
        t_cut = jnp.min(tmin_ref[...])
        w_last = jnp.maximum(iw_ref[0, n_bins - 1], 0.01)
        p = jnp.clip(yp_ref[...], 1e-7, 1.0 - 1e-7)
        yt = yt_ref[...]
        bce = -(yt * jnp.log(p) + (1.0 - yt) * jnp.log(1.0 - p))
        w = jnp.where(c_ref[...] < t_cut, w_ref[...], w_last)
        part = jnp.sum(w * bce)

        @pl.when(i == 0)
        def _init():
            out_ref[...] = jnp.zeros((1, 1), jnp.float32)

        out_ref[...] = out_ref[...] + part

        @pl.when(i == grid - 1)
        def _final():
            out_ref[...] = out_ref[...] / n_real

    return pl.pallas_call(
        body,
        grid=(grid,),
        in_specs=[
            pl.BlockSpec((_NW * _L // 128, 128), lambda i: (0, 0)),
            pl.BlockSpec((1, 128), lambda i: (0, 0)),
            pl.BlockSpec((block_rows, 128), lambda i: (i, 0)),
            pl.BlockSpec((block_rows, 128), lambda i: (i, 0)),
            pl.BlockSpec((block_rows, 128), lambda i: (i, 0)),
            pl.BlockSpec((block_rows, 128), lambda i: (i, 0)),
        ],
        out_specs=pl.BlockSpec((1, 1), lambda i: (0, 0)),
        out_shape=jax.ShapeDtypeStruct((1, 1), jnp.float32),
    )


def kernel(y_pred, y_true, interval_weights, bin_edges, conditional_info):
    n = conditional_info.shape[0]
    n_bins = bin_edges.shape[0]

    unit = _NW * _BUF
    n_padded = ((n + unit - 1) // unit) * unit
    rows = n_padded // 128
    block_rows = 1000
    while rows % block_rows:
        block_rows //= 2

    pad = n_padded - n
    cond_p = jnp.pad(conditional_info, (0, pad), constant_values=-1.0)
    yp_p = jnp.pad(y_pred.reshape(n), (0, pad))
    yt_p = jnp.pad(y_true.reshape(n), (0, pad))

    w_cand, minfail = _sc_binning(n_padded, n_bins)(
        cond_p, bin_edges, interval_weights)

    loss = _tc_loss(n, n_bins, rows, block_rows)(
        minfail.reshape(_NW * _L // 128, 128),
        interval_weights.reshape(1, n_bins),
        w_cand.reshape(rows, 128),
        cond_p.reshape(rows, 128),
        yp_p.reshape(rows, 128),
        yt_p.reshape(rows, 128),
    )
    return loss[0, 0]


# speedup vs baseline: 442.5974x; 1.1392x over previous
import functools

import jax
import jax.numpy as jnp
from jax import lax
from jax.experimental import pallas as pl
from jax.experimental.pallas import tpu as pltpu
from jax.experimental.pallas import tpu_sc as plsc

_NC = 2
_NS = 16
_NW = _NC * _NS
_L = 16

_BUF = 2000
_VPC = _BUF // _L


def _sc_binning(n_padded: int, n_bins: int):
    chunk = n_padded // _NW
    n_chunks = chunk // _BUF
    assert chunk % _BUF == 0 and n_padded % _NW == 0
    steps = []
    s = n_bins
    while s > 1:
        s //= 2
        steps.append(s)
    assert (1 << len(steps)) == n_bins

    mesh = plsc.VectorSubcoreMesh(core_axis_name="c", subcore_axis_name="s")

    assert n_chunks % 2 == 0
    unroll = 5
    assert _VPC % unroll == 0

    @functools.partial(
        pl.kernel,
        mesh=mesh,
        out_type=[
            jax.ShapeDtypeStruct((n_padded,), jnp.float32),
            jax.ShapeDtypeStruct((_NW, _L), jnp.float32),
        ],
        compiler_params=pltpu.CompilerParams(needs_layout_passes=False),
        scratch_types=[
            pltpu.VMEM((_BUF,), jnp.float32),
            pltpu.VMEM((_BUF,), jnp.float32),
            pltpu.VMEM((_BUF,), jnp.float32),
            pltpu.VMEM((_BUF,), jnp.float32),
            pltpu.VMEM((n_bins,), jnp.float32),
            pltpu.VMEM((n_bins,), jnp.float32),
            pltpu.VMEM((_L,), jnp.float32),
            pltpu.SemaphoreType.DMA,
            pltpu.SemaphoreType.DMA,
            pltpu.SemaphoreType.DMA,
            pltpu.SemaphoreType.DMA,
        ],
    )
    def sc_kernel(cond_hbm, edges_hbm, iw_hbm, w_out, minfail_out,
                  xbuf0, xbuf1, wbuf0, wbuf1, edges_v, v_v, min_v,
                  isem0, isem1, osem0, osem1):
        wid = lax.axis_index("c") * _NS + lax.axis_index("s")
        base = wid * chunk

        pltpu.sync_copy(edges_hbm, edges_v)
        pltpu.sync_copy(iw_hbm, v_v)
        for k in range(n_bins // _L):
            sl = pl.ds(k * _L, _L)
            v_v[sl] = jnp.maximum(v_v[sl], 0.01)

        big = jnp.full((_L,), jnp.inf, jnp.float32)
        top_idx = jnp.full((_L,), steps[0] - 1, jnp.int32)
        e_top = plsc.load_gather(edges_v, [top_idx])

        def one_vec(xb, wb, i, minacc):
            x = xb[pl.ds(i * _L, _L)]
            pos = jnp.where(e_top < x, steps[0], 0).astype(jnp.int32)
            for s in steps[1:]:
                probe = pos + (s - 1)
                e = plsc.load_gather(edges_v, [probe])
                pos = jnp.where(e < x, pos + s, pos)
            ej = plsc.load_gather(edges_v, [jnp.minimum(pos, n_bins - 1)])
            fail = (pos == 0) | (pos == n_bins) | (ej == x)
            fail = fail & (x >= 0.0)
            minacc = jnp.minimum(minacc, jnp.where(fail, x, big))
            widx = jnp.clip(pos - 1, 0, n_bins - 1)
            wb[pl.ds(i * _L, _L)] = plsc.load_gather(v_v, [widx])
            return minacc

        def start_in(k, xb, isem):
            pltpu.async_copy(cond_hbm.at[pl.ds(base + k * _BUF, _BUF)],
                             xb, isem)

        start_in(0, xbuf0, isem0)
        start_in(1, xbuf1, isem1)

        def half(kk, k, xb, wb, isem, osem, minacc):
            pltpu.make_async_copy(
                cond_hbm.at[pl.ds(base + k * _BUF, _BUF)], xb, isem).wait()

            @pl.when(kk >= 1)
            def _():
                pltpu.make_async_copy(
                    wb, w_out.at[pl.ds(base + k * _BUF, _BUF)], osem).wait()

            def blk(ii, acc):
                for u in range(unroll):
                    acc = one_vec(xb, wb, ii * unroll + u, acc)
                return acc

            minacc = lax.fori_loop(0, _VPC // unroll, blk, minacc)
            pltpu.async_copy(wb, w_out.at[pl.ds(base + k * _BUF, _BUF)], osem)

            @pl.when(kk <= n_chunks // 2 - 2)
            def _():
                start_in(k + 2, xb, isem)

            return minacc

        def pair(kk, minacc):
            minacc = half(kk, 2 * kk, xbuf0, wbuf0, isem0, osem0, minacc)
            minacc = half(kk, 2 * kk + 1, xbuf1, wbuf1, isem1, osem1, minacc)
            return minacc

        minacc = lax.fori_loop(0, n_chunks // 2, pair, big)

        k0 = n_chunks - 2
        pltpu.make_async_copy(
            wbuf0, w_out.at[pl.ds(base + k0 * _BUF, _BUF)], osem0).wait()
        pltpu.make_async_copy(
            wbuf1, w_out.at[pl.ds(base + (k0 + 1) * _BUF, _BUF)], osem1).wait()

        min_v[...] = minacc
        pltpu.sync_copy(min_v, minfail_out.at[wid])

    return sc_kernel


def _tc_loss(n_real: int, n_bins: int, rows: int, block_rows: int):
    grid = rows // block_rows
    assert rows % block_rows == 0

    def body(tmin_ref, iw_ref, w_ref, c_ref, yp_ref, yt_ref, out_ref):
        i = pl.program_id(0)
        t_cut = jnp.min(tmin_ref[...])
        w_last = jnp.maximum(iw_ref[0, n_bins - 1], 0.01)
        p = jnp.clip(yp_ref[...], 1e-7, 1.0 - 1e-7)
        yt = yt_ref[...]
        bce = -(yt * jnp.log(p) + (1.0 - yt) * jnp.log(1.0 - p))
        w = jnp.where(c_ref[...] < t_cut, w_ref[...], w_last)
        part = jnp.sum(w * bce)

        @pl.when(i == 0)
        def _init():
            out_ref[...] = jnp.zeros((1, 1), jnp.float32)

        out_ref[...] = out_ref[...] + part

        @pl.when(i == grid - 1)
        def _final():
            out_ref[...] = out_ref[...] / n_real

    return pl.pallas_call(
        body,
        grid=(grid,),
        in_specs=[
            pl.BlockSpec((_NW * _L // 128, 128), lambda i: (0, 0)),
            pl.BlockSpec((1, 128), lambda i: (0, 0)),
            pl.BlockSpec((block_rows, 128), lambda i: (i, 0)),
            pl.BlockSpec((block_rows, 128), lambda i: (i, 0)),
            pl.BlockSpec((block_rows, 128), lambda i: (i, 0)),
            pl.BlockSpec((block_rows, 128), lambda i: (i, 0)),
        ],
        out_specs=pl.BlockSpec((1, 1), lambda i: (0, 0)),
        out_shape=jax.ShapeDtypeStruct((1, 1), jnp.float32),
    )


def kernel(y_pred, y_true, interval_weights, bin_edges, conditional_info):
    n = conditional_info.shape[0]
    n_bins = bin_edges.shape[0]

    unit = _NW * _BUF
    n_padded = ((n + unit - 1) // unit) * unit
    rows = n_padded // 128
    block_rows = 1000
    while rows % block_rows:
        block_rows //= 2

    pad = n_padded - n
    cond_p = jnp.pad(conditional_info, (0, pad), constant_values=-1.0)
    yp_p = jnp.pad(y_pred.reshape(n), (0, pad))
    yt_p = jnp.pad(y_true.reshape(n), (0, pad))

    w_cand, minfail = _sc_binning(n_padded, n_bins)(
        cond_p, bin_edges, interval_weights)

    loss = _tc_loss(n, n_bins, rows, block_rows)(
        minfail.reshape(_NW * _L // 128, 128),
        interval_weights.reshape(1, n_bins),
        w_cand.reshape(rows, 128),
        cond_p.reshape(rows, 128),
        yp_p.reshape(rows, 128),
        yt_p.reshape(rows, 128),
    )
    return loss[0, 0]
